# Initial kernel scaffold; baseline (speedup 1.0000x reference)
import jax
import jax.numpy as jnp
from jax.experimental import pallas as pl


def kernel(pos_x, neg_x, emb_table, pair_w0, c):
    raise NotImplementedError("write your pallas kernel here")


# speedup vs baseline: 18.9611x; 18.9611x over previous
import functools

import jax
import jax.numpy as jnp
from jax import lax
from jax.experimental import pallas as pl
from jax.experimental.pallas import tpu as pltpu
from jax.experimental.pallas import tpu_sc as plsc

EMB_DIM = 32
NUM_DOMAINS = 10
NUM_NEG = 5
NUM_TILES = 32
NB = 128
LANES = 16


def _make_sc_kernel(n_rows):
    rows_per_tile = n_rows // NUM_TILES
    n_blocks = rows_per_tile // NB
    slots = NB * NUM_DOMAINS

    @functools.partial(
        pl.kernel,
        out_type=jax.ShapeDtypeStruct((n_rows,), jnp.float32),
        mesh=plsc.VectorSubcoreMesh(core_axis_name="c", subcore_axis_name="s"),
        compiler_params=pltpu.CompilerParams(use_tc_tiling_on_sc=False),
        scratch_types=[
            pltpu.VMEM((slots,), jnp.int32),
            pltpu.VMEM((slots, EMB_DIM), jnp.float32),
            pltpu.VMEM((2, LANES), jnp.float32),
            pltpu.VMEM((NB,), jnp.float32),
            pltpu.SemaphoreType.DMA,
        ],
    )
    def sc_kernel(idx_hbm, table_hbm, wc_hbm, out_hbm, idx_v, rows_v, wc_v,
                  sc_v, sem):
        wid = lax.axis_index("s") * 2 + lax.axis_index("c")
        pltpu.sync_copy(wc_hbm, wc_v)
        w_row = wc_v[0, :]
        c_row = wc_v[1, :]
        half_expw = jnp.exp(w_row) * 0.5

        def block_body(b, carry):
            blk = wid * n_blocks + b
            pltpu.sync_copy(idx_hbm.at[pl.ds(blk * slots, slots)], idx_v)
            copies = [
                pltpu.async_copy(table_hbm.at[idx_v.at[pl.ds(k * 128, 128)]],
                                 rows_v.at[pl.ds(k * 128, 128)], sem)
                for k in range(slots // 128)
            ]
            for cp in copies:
                cp.wait()

            lane = lax.iota(jnp.int32, LANES)

            def group_body(g, c2):
                def row_body(ii, acc):
                    base = (g * LANES + ii) * NUM_DOMAINS
                    v0 = rows_v[base, 0:LANES]
                    v1 = rows_v[base, LANES:EMB_DIM]
                    s0 = v0
                    s1 = v1
                    q = v0 * v0 + v1 * v1
                    for j in range(1, NUM_DOMAINS):
                        v0 = rows_v[base + j, 0:LANES]
                        v1 = rows_v[base + j, LANES:EMB_DIM]
                        s0 = s0 + v0
                        s1 = s1 + v1
                        q = q + v0 * v0 + v1 * v1
                    t = s0 * s0 + s1 * s1 - q
                    for step in (8, 4, 2, 1):
                        t = t + t.at[lane ^ step].get(
                            mode="promise_in_bounds")
                    return jnp.where(lane == ii, t, acc)

                acc = lax.fori_loop(0, LANES, row_body,
                                    jnp.zeros((LANES,), jnp.float32))
                sc_v[pl.ds(g * LANES, LANES)] = jnp.exp(
                    acc * half_expw + c_row)
                return c2

            lax.fori_loop(0, NB // LANES, group_body, 0)
            pltpu.sync_copy(sc_v, out_hbm.at[pl.ds(blk * NB, NB)])
            return carry

        lax.fori_loop(0, n_blocks, block_body, 0)

    return sc_kernel


def kernel(pos_x, neg_x, emb_table, pair_w0, c):
    b = pos_x.shape[0]
    x_all = jnp.concatenate(
        [pos_x, neg_x.reshape(b * NUM_NEG, NUM_DOMAINS)], axis=0)
    n_rows = x_all.shape[0]
    idx_flat = x_all.reshape(-1)
    wc = jnp.stack([
        jnp.broadcast_to(pair_w0[0], (LANES,)),
        jnp.broadcast_to(c[0], (LANES,)),
    ]).astype(jnp.float32)
    scores = _make_sc_kernel(n_rows)(idx_flat, emb_table, wc)
    pos_score = scores[:b]
    neg_score = scores[b:].reshape(b, NUM_NEG)
    return pos_score, neg_score
